# baseline (speedup 1.0000x reference)
import functools

import jax
import jax.numpy as jnp
from jax import lax
from jax.experimental import pallas as pl
from jax.experimental.pallas import tpu as pltpu
from jax.experimental.pallas import tpu_sc as plsc

VOCAB = 100000
EMB = 32
BATCH = 1024

_INFO = plsc.get_sparse_core_info()
_NC, _NS, _L = _INFO.num_cores, _INFO.num_subcores, _INFO.num_lanes
_NW = _NC * _NS
_B_PER_W = BATCH // _NW

_VT = 6144
_NT = (VOCAB + _VT - 1) // _VT
_NF = _NT - 1
_LAST = VOCAB - _NF * _VT
_R = 2


def _gather_body(table_hbm, idx_hbm, out_hbm, idx_v, rows_v, sem):
    wid = lax.axis_index("s") * _NC + lax.axis_index("c")
    base = wid * _B_PER_W
    pltpu.sync_copy(idx_hbm.at[pl.ds(base, _B_PER_W)], idx_v)
    pltpu.async_copy(table_hbm.at[idx_v], rows_v, sem).wait()
    pltpu.sync_copy(rows_v, out_hbm.at[pl.ds(base, _B_PER_W)])


_sc_gather = functools.partial(
    pl.kernel,
    mesh=plsc.VectorSubcoreMesh(core_axis_name="c", subcore_axis_name="s"),
    out_type=jax.ShapeDtypeStruct((BATCH, EMB), jnp.float32),
    scratch_types=[
        pltpu.VMEM((_B_PER_W,), jnp.int32),
        pltpu.VMEM((_B_PER_W, EMB), jnp.float32),
        pltpu.SemaphoreType.DMA,
    ],
    compiler_params=pltpu.CompilerParams(use_tc_tiling_on_sc=False),
)(_gather_body)


def _matmul_tile(x_ref, w_ref, b_ref):
    return (
        lax.dot_general(
            x_ref[...],
            w_ref[...],
            (((1,), (1,)), ((), ())),
            preferred_element_type=jnp.float32,
        )
        + b_ref[0]
    )


def _proj_body(x_ref, w_ref, b_ref, o_hbm, scr, sems):
    i = pl.program_id(0)
    j = lax.rem(i, _R)

    @pl.when(i >= _R)
    def _wait_prev():
        pltpu.make_async_copy(
            scr.at[j], o_hbm.at[:, pl.ds(0, _VT)], sems.at[j]
        ).wait()

    scr[j] = _matmul_tile(x_ref, w_ref, b_ref)
    pltpu.make_async_copy(
        scr.at[j], o_hbm.at[:, pl.ds(i * _VT, _VT)], sems.at[j]
    ).start()

    @pl.when(i == _NF - 1)
    def _drain():
        for jj in range(_R):
            pltpu.make_async_copy(
                scr.at[jj], o_hbm.at[:, pl.ds(0, _VT)], sems.at[jj]
            ).wait()


def _tail_body(x_ref, w_ref, b_ref, alias_ref, o_ref):
    del alias_ref
    o_ref[...] = _matmul_tile(x_ref, w_ref, b_ref)


def kernel(inputs, emb_table, W, b):
    x = _sc_gather(emb_table, inputs.astype(jnp.int32))
    bp = jnp.pad(b, (0, _NT * _VT - VOCAB)).reshape(_NT, 1, _VT)
    main = pl.pallas_call(
        _proj_body,
        grid=(_NF,),
        in_specs=[
            pl.BlockSpec((BATCH, EMB), lambda i: (0, 0)),
            pl.BlockSpec((_VT, EMB), lambda i: (i, 0)),
            pl.BlockSpec((1, 1, _VT), lambda i: (i, 0, 0)),
        ],
        out_specs=pl.BlockSpec(memory_space=pl.ANY),
        out_shape=jax.ShapeDtypeStruct((BATCH, VOCAB), jnp.float32),
        scratch_shapes=[
            pltpu.VMEM((_R, BATCH, _VT), jnp.float32),
            pltpu.SemaphoreType.DMA((_R,)),
        ],
        compiler_params=pltpu.CompilerParams(
            vmem_limit_bytes=120 * 1024 * 1024
        ),
    )(x, W, bp)
    out = pl.pallas_call(
        _tail_body,
        grid=(1,),
        in_specs=[
            pl.BlockSpec((BATCH, EMB), lambda i: (0, 0)),
            pl.BlockSpec((_VT, EMB), lambda i: (_NF, 0)),
            pl.BlockSpec((1, 1, _VT), lambda i: (_NF, 0, 0)),
            pl.BlockSpec(memory_space=pl.ANY),
        ],
        out_specs=pl.BlockSpec((BATCH, _VT), lambda i: (0, _NF)),
        out_shape=jax.ShapeDtypeStruct((BATCH, VOCAB), jnp.float32),
        input_output_aliases={3: 0},
    )(x, W, bp, main)
    return out


# speedup vs baseline: 1.0024x; 1.0024x over previous
import functools

import jax
import jax.numpy as jnp
from jax import lax
from jax.experimental import pallas as pl
from jax.experimental.pallas import tpu as pltpu
from jax.experimental.pallas import tpu_sc as plsc

VOCAB = 100000
EMB = 32
BATCH = 1024

_INFO = plsc.get_sparse_core_info()
_NC, _NS, _L = _INFO.num_cores, _INFO.num_subcores, _INFO.num_lanes
_NW = _NC * _NS
_B_PER_W = BATCH // _NW

_VT = 4096
_NT = (VOCAB + _VT - 1) // _VT
_NF = _NT - 1
_LAST = VOCAB - _NF * _VT
_R = 3


def _gather_body(table_hbm, idx_hbm, out_hbm, idx_v, rows_v, sem):
    wid = lax.axis_index("s") * _NC + lax.axis_index("c")
    base = wid * _B_PER_W
    pltpu.sync_copy(idx_hbm.at[pl.ds(base, _B_PER_W)], idx_v)
    pltpu.async_copy(table_hbm.at[idx_v], rows_v, sem).wait()
    pltpu.sync_copy(rows_v, out_hbm.at[pl.ds(base, _B_PER_W)])


_sc_gather = functools.partial(
    pl.kernel,
    mesh=plsc.VectorSubcoreMesh(core_axis_name="c", subcore_axis_name="s"),
    out_type=jax.ShapeDtypeStruct((BATCH, EMB), jnp.float32),
    scratch_types=[
        pltpu.VMEM((_B_PER_W,), jnp.int32),
        pltpu.VMEM((_B_PER_W, EMB), jnp.float32),
        pltpu.SemaphoreType.DMA,
    ],
    compiler_params=pltpu.CompilerParams(use_tc_tiling_on_sc=False),
)(_gather_body)


def _matmul_tile(x_ref, w_ref, b_ref):
    return (
        lax.dot_general(
            x_ref[...],
            w_ref[...],
            (((1,), (1,)), ((), ())),
            preferred_element_type=jnp.float32,
        )
        + b_ref[0]
    )


def _proj_body(x_ref, w_ref, b_ref, o_hbm, scr, sems):
    i = pl.program_id(0)
    j = lax.rem(i, _R)

    @pl.when(i >= _R)
    def _wait_prev():
        pltpu.make_async_copy(
            scr.at[j], o_hbm.at[:, pl.ds(0, _VT)], sems.at[j]
        ).wait()

    scr[j] = _matmul_tile(x_ref, w_ref, b_ref)
    pltpu.make_async_copy(
        scr.at[j], o_hbm.at[:, pl.ds(i * _VT, _VT)], sems.at[j]
    ).start()

    @pl.when(i == _NF - 1)
    def _drain():
        for jj in range(_R):
            pltpu.make_async_copy(
                scr.at[jj], o_hbm.at[:, pl.ds(0, _VT)], sems.at[jj]
            ).wait()


def _tail_body(x_ref, w_ref, b_ref, alias_ref, o_ref):
    del alias_ref
    o_ref[...] = _matmul_tile(x_ref, w_ref, b_ref)


def kernel(inputs, emb_table, W, b):
    x = _sc_gather(emb_table, inputs.astype(jnp.int32))
    bp = jnp.pad(b, (0, _NT * _VT - VOCAB)).reshape(_NT, 1, _VT)
    main = pl.pallas_call(
        _proj_body,
        grid=(_NF,),
        in_specs=[
            pl.BlockSpec((BATCH, EMB), lambda i: (0, 0)),
            pl.BlockSpec((_VT, EMB), lambda i: (i, 0)),
            pl.BlockSpec((1, 1, _VT), lambda i: (i, 0, 0)),
        ],
        out_specs=pl.BlockSpec(memory_space=pl.ANY),
        out_shape=jax.ShapeDtypeStruct((BATCH, VOCAB), jnp.float32),
        scratch_shapes=[
            pltpu.VMEM((_R, BATCH, _VT), jnp.float32),
            pltpu.SemaphoreType.DMA((_R,)),
        ],
        compiler_params=pltpu.CompilerParams(
            vmem_limit_bytes=120 * 1024 * 1024
        ),
    )(x, W, bp)
    out = pl.pallas_call(
        _tail_body,
        grid=(1,),
        in_specs=[
            pl.BlockSpec((BATCH, EMB), lambda i: (0, 0)),
            pl.BlockSpec((_VT, EMB), lambda i: (_NF, 0)),
            pl.BlockSpec((1, 1, _VT), lambda i: (_NF, 0, 0)),
            pl.BlockSpec(memory_space=pl.ANY),
        ],
        out_specs=pl.BlockSpec((BATCH, _VT), lambda i: (0, _NF)),
        out_shape=jax.ShapeDtypeStruct((BATCH, VOCAB), jnp.float32),
        input_output_aliases={3: 0},
    )(x, W, bp, main)
    return out


# speedup vs baseline: 1.0071x; 1.0047x over previous
import functools

import jax
import jax.numpy as jnp
from jax import lax
from jax.experimental import pallas as pl
from jax.experimental.pallas import tpu as pltpu
from jax.experimental.pallas import tpu_sc as plsc

VOCAB = 100000
EMB = 32
BATCH = 1024

_INFO = plsc.get_sparse_core_info()
_NC, _NS, _L = _INFO.num_cores, _INFO.num_subcores, _INFO.num_lanes
_NW = _NC * _NS
_B_PER_W = BATCH // _NW

_VT = 4096
_NT = (VOCAB + _VT - 1) // _VT
_NF = _NT - 1
_LAST = VOCAB - _NF * _VT
_R = 3


_PACK = 128 // EMB


def _gather_body(table_hbm, idx_hbm, out_hbm, idx_v, rows_v, sem):
    wid = lax.axis_index("s") * _NC + lax.axis_index("c")
    base = wid * _B_PER_W
    pltpu.sync_copy(idx_hbm.at[pl.ds(base, _B_PER_W)], idx_v)
    pltpu.async_copy(table_hbm.at[idx_v], rows_v, sem).wait()
    pltpu.sync_copy(rows_v, out_hbm.at[pl.ds(base, _B_PER_W)])


_sc_gather = functools.partial(
    pl.kernel,
    mesh=plsc.VectorSubcoreMesh(core_axis_name="c", subcore_axis_name="s"),
    out_type=jax.ShapeDtypeStruct((BATCH, _PACK * EMB), jnp.float32),
    scratch_types=[
        pltpu.VMEM((_B_PER_W,), jnp.int32),
        pltpu.VMEM((_B_PER_W, _PACK * EMB), jnp.float32),
        pltpu.SemaphoreType.DMA,
    ],
)(_gather_body)


def _extract_x(x4_ref, sub_ref):
    sub = sub_ref[:, :EMB]
    x = jnp.zeros((BATCH, EMB), jnp.float32)
    for q in range(_PACK):
        x = x + jnp.where(sub == q, x4_ref[:, q * EMB:(q + 1) * EMB], 0.0)
    return x


def _matmul_tile(x, w_ref, b_ref):
    return (
        lax.dot_general(
            x,
            w_ref[...],
            (((1,), (1,)), ((), ())),
            preferred_element_type=jnp.float32,
        )
        + b_ref[0]
    )


def _proj_body(x_ref, sub_ref, w_ref, b_ref, o_hbm, scr, sems):
    i = pl.program_id(0)
    j = lax.rem(i, _R)

    @pl.when(i >= _R)
    def _wait_prev():
        pltpu.make_async_copy(
            scr.at[j], o_hbm.at[:, pl.ds(0, _VT)], sems.at[j]
        ).wait()

    scr[j] = _matmul_tile(_extract_x(x_ref, sub_ref), w_ref, b_ref)
    pltpu.make_async_copy(
        scr.at[j], o_hbm.at[:, pl.ds(i * _VT, _VT)], sems.at[j]
    ).start()

    @pl.when(i == _NF - 1)
    def _drain():
        for jj in range(_R):
            pltpu.make_async_copy(
                scr.at[jj], o_hbm.at[:, pl.ds(0, _VT)], sems.at[jj]
            ).wait()


def _tail_body(x_ref, sub_ref, w_ref, b_ref, alias_ref, o_ref):
    del alias_ref
    o_ref[...] = _matmul_tile(_extract_x(x_ref, sub_ref), w_ref, b_ref)


def kernel(inputs, emb_table, W, b):
    idx = inputs.astype(jnp.int32)
    x4 = _sc_gather(emb_table.reshape(VOCAB // _PACK, _PACK * EMB),
                    idx // _PACK)
    sub = jnp.broadcast_to((idx % _PACK)[:, None], (BATCH, _PACK * EMB))
    bp = jnp.pad(b, (0, _NT * _VT - VOCAB)).reshape(_NT, 1, _VT)
    main = pl.pallas_call(
        _proj_body,
        grid=(_NF,),
        in_specs=[
            pl.BlockSpec((BATCH, _PACK * EMB), lambda i: (0, 0)),
            pl.BlockSpec((BATCH, _PACK * EMB), lambda i: (0, 0)),
            pl.BlockSpec((_VT, EMB), lambda i: (i, 0)),
            pl.BlockSpec((1, 1, _VT), lambda i: (i, 0, 0)),
        ],
        out_specs=pl.BlockSpec(memory_space=pl.ANY),
        out_shape=jax.ShapeDtypeStruct((BATCH, VOCAB), jnp.float32),
        scratch_shapes=[
            pltpu.VMEM((_R, BATCH, _VT), jnp.float32),
            pltpu.SemaphoreType.DMA((_R,)),
        ],
        compiler_params=pltpu.CompilerParams(
            vmem_limit_bytes=120 * 1024 * 1024
        ),
    )(x4, sub, W, bp)
    out = pl.pallas_call(
        _tail_body,
        grid=(1,),
        in_specs=[
            pl.BlockSpec((BATCH, _PACK * EMB), lambda i: (0, 0)),
            pl.BlockSpec((BATCH, _PACK * EMB), lambda i: (0, 0)),
            pl.BlockSpec((_VT, EMB), lambda i: (_NF, 0)),
            pl.BlockSpec((1, 1, _VT), lambda i: (_NF, 0, 0)),
            pl.BlockSpec(memory_space=pl.ANY),
        ],
        out_specs=pl.BlockSpec((BATCH, _VT), lambda i: (0, _NF)),
        out_shape=jax.ShapeDtypeStruct((BATCH, VOCAB), jnp.float32),
        input_output_aliases={4: 0},
    )(x4, sub, W, bp, main)
    return out


# speedup vs baseline: 1.0134x; 1.0063x over previous
import functools

import jax
import jax.numpy as jnp
from jax import lax
from jax.experimental import pallas as pl
from jax.experimental.pallas import tpu as pltpu
from jax.experimental.pallas import tpu_sc as plsc

VOCAB = 100000
EMB = 32
BATCH = 1024

_INFO = plsc.get_sparse_core_info()
_NC, _NS, _L = _INFO.num_cores, _INFO.num_subcores, _INFO.num_lanes
_NW = _NC * _NS
_B_PER_W = BATCH // _NW

_VT = 6144
_NT = (VOCAB + _VT - 1) // _VT


_PACK = 128 // EMB


def _gather_body(table_hbm, idx_hbm, out_hbm, idx_v, rows_v, sem):
    wid = lax.axis_index("s") * _NC + lax.axis_index("c")
    base = wid * _B_PER_W
    pltpu.sync_copy(idx_hbm.at[pl.ds(base, _B_PER_W)], idx_v)
    pltpu.async_copy(table_hbm.at[idx_v], rows_v, sem).wait()
    pltpu.sync_copy(rows_v, out_hbm.at[pl.ds(base, _B_PER_W)])


_sc_gather = functools.partial(
    pl.kernel,
    mesh=plsc.VectorSubcoreMesh(core_axis_name="c", subcore_axis_name="s"),
    out_type=jax.ShapeDtypeStruct((BATCH, _PACK * EMB), jnp.float32),
    scratch_types=[
        pltpu.VMEM((_B_PER_W,), jnp.int32),
        pltpu.VMEM((_B_PER_W, _PACK * EMB), jnp.float32),
        pltpu.SemaphoreType.DMA,
    ],
)(_gather_body)


def _extract_x(x4_ref, sub_ref):
    sub = sub_ref[:, :EMB]
    x = jnp.zeros((BATCH, EMB), jnp.float32)
    for q in range(_PACK):
        x = x + jnp.where(sub == q, x4_ref[:, q * EMB:(q + 1) * EMB], 0.0)
    return x


def _matmul_tile(x, w_ref, b_ref):
    return (
        lax.dot_general(
            x,
            w_ref[...],
            (((1,), (1,)), ((), ())),
            preferred_element_type=jnp.float32,
        )
        + b_ref[0]
    )


def _proj_body(x_ref, sub_ref, w_ref, b_ref, o_ref):
    o_ref[...] = _matmul_tile(_extract_x(x_ref, sub_ref), w_ref, b_ref)


def kernel(inputs, emb_table, W, b):
    idx = inputs.astype(jnp.int32)
    x4 = _sc_gather(emb_table.reshape(VOCAB // _PACK, _PACK * EMB),
                    idx // _PACK)
    sub = jnp.broadcast_to((idx % _PACK)[:, None], (BATCH, _PACK * EMB))
    bp = jnp.pad(b, (0, _NT * _VT - VOCAB)).reshape(_NT, 1, _VT)
    out = pl.pallas_call(
        _proj_body,
        grid=(_NT,),
        in_specs=[
            pl.BlockSpec((BATCH, _PACK * EMB), lambda i: (0, 0)),
            pl.BlockSpec((BATCH, _PACK * EMB), lambda i: (0, 0)),
            pl.BlockSpec((_VT, EMB), lambda i: (i, 0)),
            pl.BlockSpec((1, 1, _VT), lambda i: (i, 0, 0)),
        ],
        out_specs=pl.BlockSpec((BATCH, _VT), lambda i: (0, i)),
        out_shape=jax.ShapeDtypeStruct((BATCH, VOCAB), jnp.float32),
        compiler_params=pltpu.CompilerParams(
            vmem_limit_bytes=120 * 1024 * 1024
        ),
    )(x4, sub, W, bp)
    return out


# speedup vs baseline: 1.0138x; 1.0004x over previous
import functools

import jax
import jax.numpy as jnp
from jax import lax
from jax.experimental import pallas as pl
from jax.experimental.pallas import tpu as pltpu
from jax.experimental.pallas import tpu_sc as plsc

VOCAB = 100000
EMB = 32
BATCH = 1024

_INFO = plsc.get_sparse_core_info()
_NC, _NS, _L = _INFO.num_cores, _INFO.num_subcores, _INFO.num_lanes
_NW = _NC * _NS
_B_PER_W = BATCH // _NW

_VT = 6144
_NT = (VOCAB + _VT - 1) // _VT


_PACK = 128 // EMB


def _gather_body(table_hbm, idx_hbm, out_hbm, idx_v, rows_v, sem):
    wid = lax.axis_index("s") * _NC + lax.axis_index("c")
    base = wid * _B_PER_W
    pltpu.sync_copy(idx_hbm.at[pl.ds(base, _B_PER_W)], idx_v)
    pltpu.async_copy(table_hbm.at[idx_v], rows_v, sem).wait()
    pltpu.sync_copy(rows_v, out_hbm.at[pl.ds(base, _B_PER_W)])


_sc_gather = functools.partial(
    pl.kernel,
    mesh=plsc.VectorSubcoreMesh(core_axis_name="c", subcore_axis_name="s"),
    out_type=jax.ShapeDtypeStruct((BATCH, _PACK * EMB), jnp.float32),
    scratch_types=[
        pltpu.VMEM((_B_PER_W,), jnp.int32),
        pltpu.VMEM((_B_PER_W, _PACK * EMB), jnp.float32),
        pltpu.SemaphoreType.DMA,
    ],
)(_gather_body)


def _extract_x(x4_ref, sub_ref):
    sub = sub_ref[...]
    x = jnp.zeros((BATCH, EMB), jnp.float32)
    for q in range(_PACK):
        x = x + jnp.where(sub == q, x4_ref[:, q * EMB:(q + 1) * EMB], 0.0)
    return x


def _matmul_tile(x, w_ref, b_ref):
    return (
        lax.dot_general(
            x,
            w_ref[...],
            (((1,), (1,)), ((), ())),
            preferred_element_type=jnp.float32,
        )
        + b_ref[0]
    )


def _proj_body(x_ref, sub_ref, w_ref, b_ref, o_ref):
    o_ref[...] = _matmul_tile(_extract_x(x_ref, sub_ref), w_ref, b_ref)


def kernel(inputs, emb_table, W, b):
    idx = inputs.astype(jnp.int32)
    x4 = _sc_gather(emb_table.reshape(VOCAB // _PACK, _PACK * EMB),
                    idx // _PACK)
    sub = jnp.broadcast_to((idx % _PACK)[:, None], (BATCH, EMB))
    bp = jnp.pad(b, (0, _NT * _VT - VOCAB)).reshape(_NT, 1, _VT)
    out = pl.pallas_call(
        _proj_body,
        grid=(_NT,),
        in_specs=[
            pl.BlockSpec((BATCH, _PACK * EMB), lambda i: (0, 0)),
            pl.BlockSpec((BATCH, EMB), lambda i: (0, 0)),
            pl.BlockSpec((_VT, EMB), lambda i: (i, 0)),
            pl.BlockSpec((1, 1, _VT), lambda i: (i, 0, 0)),
        ],
        out_specs=pl.BlockSpec((BATCH, _VT), lambda i: (0, i)),
        out_shape=jax.ShapeDtypeStruct((BATCH, VOCAB), jnp.float32),
        compiler_params=pltpu.CompilerParams(
            vmem_limit_bytes=120 * 1024 * 1024
        ),
    )(x4, sub, W, bp)
    return out
